# Initial kernel scaffold; baseline (speedup 1.0000x reference)
import jax
import jax.numpy as jnp
from jax.experimental import pallas as pl


def kernel(indices, table, prompt_no):
    raise NotImplementedError("write your pallas kernel here")


# speedup vs baseline: 1.0255x; 1.0255x over previous
import functools

import jax
import jax.numpy as jnp
from jax import lax
from jax.experimental import pallas as pl
from jax.experimental.pallas import tpu as pltpu
from jax.experimental.pallas import tpu_sc as plsc

NC = 2
NS = 16
NW = NC * NS
LANES = 16
NBUF = 8
K = 4


def _pn_mean(prompt_no):
    def body(p_ref, o_ref):
        o_ref[...] = jnp.mean(p_ref[...], axis=0)

    return pl.pallas_call(
        body,
        out_shape=jax.ShapeDtypeStruct(prompt_no.shape[1:], jnp.float32),
    )(prompt_no)


def _sc_gather_add(idx_r, table, pn):
    nblk = idx_r.shape[1]
    lr = idx_r.shape[2]
    d = table.shape[1]
    nq = d // LANES
    niter = nblk // NBUF

    mesh = plsc.VectorSubcoreMesh(core_axis_name="c", subcore_axis_name="s")

    @functools.partial(
        pl.kernel,
        out_type=jax.ShapeDtypeStruct((NW, nblk, lr, d), jnp.float32),
        mesh=mesh,
        scratch_types=[
            pltpu.VMEM((nblk, lr), jnp.int32),
            pltpu.VMEM((lr, d), jnp.float32),
            pltpu.VMEM((NBUF, lr, d), jnp.float32),
            pltpu.VMEM((NBUF, lr, d), jnp.float32),
            pltpu.SemaphoreType.DMA((NBUF,)),
            pltpu.SemaphoreType.DMA((NBUF,)),
        ],
        compiler_params=pltpu.CompilerParams(use_tc_tiling_on_sc=False),
    )
    def k(idx_hbm, table_hbm, pn_hbm, out_hbm, idx_v, pn_v, gbuf, obuf,
          gsem, osem):
        wid = lax.axis_index("s") * NC + lax.axis_index("c")
        pltpu.sync_copy(idx_hbm.at[wid], idx_v)
        pltpu.sync_copy(pn_hbm, pn_v)

        for s in range(NBUF):
            pltpu.async_copy(table_hbm.at[idx_v.at[s]], gbuf.at[s],
                             gsem.at[s])

        def outer(i, carry):
            base = i * NBUF
            for half in range(2):
                slots = [half * K + j for j in range(K)]
                for s in slots:
                    pltpu.make_async_copy(
                        table_hbm.at[idx_v.at[0]], gbuf.at[s],
                        gsem.at[s]).wait()

                    @pl.when(i > 0)
                    def _(s=s):
                        pltpu.make_async_copy(
                            obuf.at[s], out_hbm.at[0, 0],
                            osem.at[s]).wait()

                def row_body(r, c2):
                    for q in range(nq):
                        col = q * LANES
                        pnq = pn_v[r, pl.ds(col, LANES)]
                        for s in slots:
                            obuf[s, r, pl.ds(col, LANES)] = (
                                gbuf[s, r, pl.ds(col, LANES)] + pnq)
                    return c2

                lax.fori_loop(0, lr, row_body, 0, unroll=1)

                for j, s in enumerate(slots):
                    b = base + half * K + j
                    pltpu.async_copy(obuf.at[s], out_hbm.at[wid, b],
                                     osem.at[s])

                    @pl.when(i < niter - 1)
                    def _(b=b, s=s):
                        pltpu.async_copy(
                            table_hbm.at[idx_v.at[b + NBUF]],
                            gbuf.at[s], gsem.at[s])
            return carry

        lax.fori_loop(0, niter, outer, 0)

        for s in range(NBUF):
            pltpu.make_async_copy(obuf.at[s], out_hbm.at[0, 0],
                                  osem.at[s]).wait()

    return k(idx_r, table, pn)


def kernel(indices, table, prompt_no):
    b, l = indices.shape
    d = table.shape[1]
    pn = _pn_mean(prompt_no)
    idx_r = indices.astype(jnp.int32).reshape(NW, (b * l) // (NW * l), l)
    out = _sc_gather_add(idx_r, table, pn)
    return out.reshape(b, l, d)


# speedup vs baseline: 1.0266x; 1.0011x over previous
import functools

import jax
import jax.numpy as jnp
from jax import lax
from jax.experimental import pallas as pl
from jax.experimental.pallas import tpu as pltpu
from jax.experimental.pallas import tpu_sc as plsc

NC = 2
NS = 16
NW = NC * NS
LANES = 16
NBUF = 8
K = 4


def _pn_mean(prompt_no):
    def body(p_ref, o_ref):
        o_ref[...] = jnp.mean(p_ref[...], axis=0)

    return pl.pallas_call(
        body,
        out_shape=jax.ShapeDtypeStruct(prompt_no.shape[1:], jnp.float32),
    )(prompt_no)


def _sc_gather_add(idx, table, pn):
    batch = idx.shape[0]
    lr = idx.shape[1]
    d = table.shape[1]
    nq = d // LANES
    nblk = batch // NW
    niter = nblk // NBUF

    mesh = plsc.VectorSubcoreMesh(core_axis_name="c", subcore_axis_name="s")

    @functools.partial(
        pl.kernel,
        out_type=jax.ShapeDtypeStruct((batch, lr, d), jnp.float32),
        mesh=mesh,
        scratch_types=[
            pltpu.VMEM((nblk, lr), jnp.int32),
            pltpu.VMEM((lr, d), jnp.float32),
            pltpu.VMEM((NBUF, lr, d), jnp.float32),
            pltpu.VMEM((NBUF, lr, d), jnp.float32),
            pltpu.SemaphoreType.DMA((NBUF,)),
            pltpu.SemaphoreType.DMA((NBUF,)),
        ],
        compiler_params=pltpu.CompilerParams(use_tc_tiling_on_sc=False),
    )
    def k(idx_hbm, table_hbm, pn_hbm, out_hbm, idx_v, pn_v, gbuf, obuf,
          gsem, osem):
        wid = lax.axis_index("s") * NC + lax.axis_index("c")
        row0 = wid * nblk
        pltpu.sync_copy(idx_hbm.at[pl.ds(row0, nblk)], idx_v)
        pltpu.sync_copy(pn_hbm, pn_v)

        for s in range(NBUF):
            pltpu.async_copy(table_hbm.at[idx_v.at[s]], gbuf.at[s],
                             gsem.at[s])

        def outer(i, carry):
            base = i * NBUF
            for half in range(2):
                slots = [half * K + j for j in range(K)]
                for s in slots:
                    pltpu.make_async_copy(
                        table_hbm.at[idx_v.at[0]], gbuf.at[s],
                        gsem.at[s]).wait()

                    @pl.when(i > 0)
                    def _(s=s):
                        pltpu.make_async_copy(
                            obuf.at[s], out_hbm.at[0],
                            osem.at[s]).wait()

                def row_body(r, c2):
                    for q in range(nq):
                        col = q * LANES
                        pnq = pn_v[r, pl.ds(col, LANES)]
                        for s in slots:
                            obuf[s, r, pl.ds(col, LANES)] = (
                                gbuf[s, r, pl.ds(col, LANES)] + pnq)
                    return c2

                lax.fori_loop(0, lr, row_body, 0, unroll=1)

                for j, s in enumerate(slots):
                    b = base + half * K + j
                    pltpu.async_copy(obuf.at[s], out_hbm.at[row0 + b],
                                     osem.at[s])

                    @pl.when(i < niter - 1)
                    def _(b=b, s=s):
                        pltpu.async_copy(
                            table_hbm.at[idx_v.at[b + NBUF]],
                            gbuf.at[s], gsem.at[s])
            return carry

        lax.fori_loop(0, niter, outer, 0)

        for s in range(NBUF):
            pltpu.make_async_copy(obuf.at[s], out_hbm.at[0],
                                  osem.at[s]).wait()

    return k(idx, table, pn)


def kernel(indices, table, prompt_no):
    pn = _pn_mean(prompt_no)
    return _sc_gather_add(indices.astype(jnp.int32), table, pn)
